# baseline (speedup 1.0000x reference)
import jax
import jax.numpy as jnp
from jax.experimental import pallas as pl

B, L, K = 1024, 50, 9
D_IN, D_OUT, P = 128, 200, 150
BB = 128
LP = 56
T = BB * LP
CH = 128


def _comb_block(tab_ref, w_ref, out_ref):
    out_ref[...] = jax.lax.dot_general(
        tab_ref[...], w_ref[...], (((1,), (0,)), ((), ())),
        preferred_element_type=jnp.float32)


def _attn_block(x_ref, idx_ref, comb_ref, tab_ref, out_ref):
    x = x_ref[...].reshape(T, D_IN)
    idx = idx_ref[...].reshape(T, K)
    comb = comb_ref[...]
    tab = tab_ref[...]

    s = jax.lax.dot_general(x, comb, (((1,), (1,)), ((), ())),
                            preferred_element_type=jnp.float32)

    iota = jax.lax.broadcasted_iota(jnp.int32, (CH, P), 1)
    chunks = []
    for c in range(T // CH):
        sc = jax.lax.slice(s, (c * CH, 0), ((c + 1) * CH, P))
        idxc = jax.lax.slice(idx, (c * CH, 0), ((c + 1) * CH, K))
        cnt = (idxc[:, 0:1] == iota).astype(jnp.float32)
        for k in range(1, K):
            cnt += (idxc[:, k:k + 1] == iota).astype(jnp.float32)
        m = jnp.max(sc, axis=1, keepdims=True)
        e = cnt * jnp.exp(sc - m)
        chunks.append(e / jnp.sum(e, axis=1, keepdims=True))
    probs = jnp.concatenate(chunks, axis=0)

    out = jax.lax.dot_general(
        probs, tab, (((1,), (0,)), ((), ())),
        preferred_element_type=jnp.float32)
    out_ref[...] = out.reshape(BB, LP, D_OUT)


@jax.jit
def kernel(input_context, cand_idx, pos_table, W):
    idx = cand_idx.astype(jnp.int32)

    comb = pl.pallas_call(
        _comb_block,
        out_shape=jax.ShapeDtypeStruct((P, D_IN), jnp.float32),
    )(pos_table, W)

    return pl.pallas_call(
        _attn_block,
        grid=(B // BB,),
        in_specs=[
            pl.BlockSpec((BB, LP, D_IN), lambda i: (i, 0, 0)),
            pl.BlockSpec((BB, LP, K), lambda i: (i, 0, 0)),
            pl.BlockSpec((P, D_IN), lambda i: (0, 0)),
            pl.BlockSpec((P, D_OUT), lambda i: (0, 0)),
        ],
        out_specs=pl.BlockSpec((BB, LP, D_OUT), lambda i: (i, 0, 0)),
        out_shape=jax.ShapeDtypeStruct((B, L, D_OUT), jnp.float32),
    )(input_context, idx, comb, pos_table)


# speedup vs baseline: 1.1060x; 1.1060x over previous
import jax
import jax.numpy as jnp
from jax.experimental import pallas as pl

B, L, K = 1024, 50, 9
D_IN, D_OUT, P = 128, 200, 150
BB = 128
LP = 56
T = BB * LP
BBC = 4
CH = BBC * LP


def _comb_block(tab_ref, w_ref, out_ref):
    out_ref[...] = jax.lax.dot_general(
        tab_ref[...], w_ref[...], (((1,), (0,)), ((), ())),
        preferred_element_type=jnp.float32)


def _attn_block(x_ref, idx_ref, comb_ref, tab_ref, out_ref):
    comb = comb_ref[...]
    tab = tab_ref[...]

    iota = jax.lax.broadcasted_iota(jnp.int32, (CH, P), 1)
    for c in range(BB // BBC):
        xc = x_ref[c * BBC:(c + 1) * BBC].reshape(CH, D_IN)
        idxc = idx_ref[c * BBC:(c + 1) * BBC].reshape(CH, K)
        s = jax.lax.dot_general(xc, comb, (((1,), (1,)), ((), ())),
                                preferred_element_type=jnp.float32)
        cnt = (idxc[:, 0:1] == iota).astype(jnp.float32)
        for k in range(1, K):
            cnt += (idxc[:, k:k + 1] == iota).astype(jnp.float32)
        m = jnp.max(s, axis=1, keepdims=True)
        e = cnt * jnp.exp(s - m)
        probs = e / jnp.sum(e, axis=1, keepdims=True)
        out = jax.lax.dot_general(
            probs, tab, (((1,), (0,)), ((), ())),
            preferred_element_type=jnp.float32)
        out_ref[c * BBC:(c + 1) * BBC] = out.reshape(BBC, LP, D_OUT)


@jax.jit
def kernel(input_context, cand_idx, pos_table, W):
    idx = cand_idx.astype(jnp.int32)

    comb = pl.pallas_call(
        _comb_block,
        out_shape=jax.ShapeDtypeStruct((P, D_IN), jnp.float32),
    )(pos_table, W)

    return pl.pallas_call(
        _attn_block,
        grid=(B // BB,),
        in_specs=[
            pl.BlockSpec((BB, LP, D_IN), lambda i: (i, 0, 0)),
            pl.BlockSpec((BB, LP, K), lambda i: (i, 0, 0)),
            pl.BlockSpec((P, D_IN), lambda i: (0, 0)),
            pl.BlockSpec((P, D_OUT), lambda i: (0, 0)),
        ],
        out_specs=pl.BlockSpec((BB, LP, D_OUT), lambda i: (i, 0, 0)),
        out_shape=jax.ShapeDtypeStruct((B, L, D_OUT), jnp.float32),
    )(input_context, idx, comb, pos_table)


# speedup vs baseline: 1.3162x; 1.1900x over previous
import jax
import jax.numpy as jnp
from jax.experimental import pallas as pl

B, L, K = 1024, 50, 9
D_IN, D_OUT, P = 128, 200, 150
BB = 128
LP = 56
T = BB * LP
BBC = 128
CH = BBC * LP


def _comb_block(tab_ref, w_ref, out_ref):
    out_ref[...] = jax.lax.dot_general(
        tab_ref[...], w_ref[...], (((1,), (0,)), ((), ())),
        preferred_element_type=jnp.float32)


def _attn_block(x_ref, idx_ref, comb_ref, tab_ref, out_ref):
    comb = comb_ref[...]
    tab = tab_ref[...]

    iota = jax.lax.broadcasted_iota(jnp.int32, (CH, P), 1)
    for c in range(BB // BBC):
        xc = x_ref[c * BBC:(c + 1) * BBC].reshape(CH, D_IN)
        idxc = idx_ref[c * BBC:(c + 1) * BBC].reshape(CH, K)
        s = jax.lax.dot_general(xc, comb, (((1,), (1,)), ((), ())),
                                preferred_element_type=jnp.float32)
        cnt = (idxc[:, 0:1] == iota).astype(jnp.float32)
        for k in range(1, K):
            cnt += (idxc[:, k:k + 1] == iota).astype(jnp.float32)
        m = jnp.max(s, axis=1, keepdims=True)
        e = cnt * jnp.exp(s - m)
        probs = e / jnp.sum(e, axis=1, keepdims=True)
        out = jax.lax.dot_general(
            probs, tab, (((1,), (0,)), ((), ())),
            preferred_element_type=jnp.float32)
        out_ref[c * BBC:(c + 1) * BBC] = out.reshape(BBC, LP, D_OUT)


@jax.jit
def kernel(input_context, cand_idx, pos_table, W):
    idx = cand_idx.astype(jnp.int32)

    comb = pl.pallas_call(
        _comb_block,
        out_shape=jax.ShapeDtypeStruct((P, D_IN), jnp.float32),
    )(pos_table, W)

    return pl.pallas_call(
        _attn_block,
        grid=(B // BB,),
        in_specs=[
            pl.BlockSpec((BB, LP, D_IN), lambda i: (i, 0, 0)),
            pl.BlockSpec((BB, LP, K), lambda i: (i, 0, 0)),
            pl.BlockSpec((P, D_IN), lambda i: (0, 0)),
            pl.BlockSpec((P, D_OUT), lambda i: (0, 0)),
        ],
        out_specs=pl.BlockSpec((BB, LP, D_OUT), lambda i: (i, 0, 0)),
        out_shape=jax.ShapeDtypeStruct((B, L, D_OUT), jnp.float32),
    )(input_context, idx, comb, pos_table)


# speedup vs baseline: 1.3202x; 1.0030x over previous
import jax
import jax.numpy as jnp
from jax.experimental import pallas as pl

B, L, K = 1024, 50, 9
D_IN, D_OUT, P = 128, 200, 150
BB = 128
LP = 56
T = BB * LP


def _comb_block(tab_ref, w_ref, out_ref):
    out_ref[...] = jax.lax.dot_general(
        tab_ref[...], w_ref[...], (((1,), (0,)), ((), ())),
        preferred_element_type=jnp.float32)


def _attn_block(x_ref, idx_ref, comb_ref, tab_ref, out_ref):
    x = x_ref[...].reshape(T, D_IN)
    idx = idx_ref[...].reshape(T, K)
    comb = comb_ref[...]
    tab = tab_ref[...]

    s = jax.lax.dot_general(x, comb, (((1,), (1,)), ((), ())),
                            preferred_element_type=jnp.float32)

    iota = jax.lax.broadcasted_iota(jnp.int32, (T, P), 1)
    cnt = (idx[:, 0:1] == iota).astype(jnp.float32)
    for k in range(1, K):
        cnt += (idx[:, k:k + 1] == iota).astype(jnp.float32)

    m = jnp.max(s, axis=1, keepdims=True)
    e = cnt * jnp.exp(s - m)
    probs = e / jnp.sum(e, axis=1, keepdims=True)

    out = jax.lax.dot_general(
        probs, tab, (((1,), (0,)), ((), ())),
        preferred_element_type=jnp.float32)
    out_ref[...] = out.reshape(BB, LP, D_OUT)


@jax.jit
def kernel(input_context, cand_idx, pos_table, W):
    idx = cand_idx.astype(jnp.int32)

    comb = pl.pallas_call(
        _comb_block,
        out_shape=jax.ShapeDtypeStruct((P, D_IN), jnp.float32),
    )(pos_table, W)

    return pl.pallas_call(
        _attn_block,
        grid=(B // BB,),
        in_specs=[
            pl.BlockSpec((BB, LP, D_IN), lambda i: (i, 0, 0)),
            pl.BlockSpec((BB, LP, K), lambda i: (i, 0, 0)),
            pl.BlockSpec((P, D_IN), lambda i: (0, 0)),
            pl.BlockSpec((P, D_OUT), lambda i: (0, 0)),
        ],
        out_specs=pl.BlockSpec((BB, LP, D_OUT), lambda i: (i, 0, 0)),
        out_shape=jax.ShapeDtypeStruct((B, L, D_OUT), jnp.float32),
    )(input_context, idx, comb, pos_table)
